# speedup vs baseline: 1.0885x; 1.0885x over previous
import jax
import jax.numpy as jnp
from jax.experimental import pallas as pl

N_NEWS = 10000
N_ENTITY = 30000
N_USERS = 4096
N_FACTORS = 8
N_REL = 40
D = 100
N_NEIGH = 20


def _combine_body(user_agg_ref, mod_ref, out_ref):
    out_ref[...] = user_agg_ref[...] * mod_ref[...]


def kernel(user_emb, all_embedding, entity_emb, relation_emb, latent_emb, weight,
           disen_weight_att, interact_vals, news_entities, news_relations,
           neigh_entities, neigh_relations, interact_rows, interact_cols):
    news_neigh_ent = jnp.take(entity_emb, news_entities, axis=0)
    news_agg = jnp.mean(news_neigh_ent, axis=1) + relation_emb[0][None, :]
    node_news = news_agg + all_embedding[:N_NEWS]

    neigh_ent_emb = jnp.take(all_embedding, neigh_entities, axis=0)
    neigh_rel_emb = jnp.take(relation_emb, neigh_relations, axis=0)
    entity_agg = jnp.mean(neigh_rel_emb + neigh_ent_emb, axis=1)
    node_ent = entity_agg + all_embedding

    node_emb = jnp.concatenate([node_news, node_ent], axis=0)

    gathered = jnp.take(node_emb, interact_cols, axis=0) * interact_vals[:, None]
    user_agg = jax.ops.segment_sum(gathered, interact_rows, num_segments=N_USERS)

    score = jax.nn.softmax(user_emb @ latent_emb.T, axis=1)
    disen_weight = jax.nn.softmax(disen_weight_att, axis=-1) @ weight
    mod = 1.0 + score @ disen_weight

    user_out = pl.pallas_call(
        _combine_body,
        out_shape=jax.ShapeDtypeStruct((N_USERS, D), jnp.float32),
    )(user_agg, mod)
    return (node_emb, user_out)


# speedup vs baseline: 3.8457x; 3.5330x over previous
import functools

import jax
import jax.numpy as jnp
from jax import lax
from jax.experimental import pallas as pl
from jax.experimental.pallas import tpu as pltpu
from jax.experimental.pallas import tpu_sc as plsc

N_NEWS = 10000
N_ENTITY = 30000
N_NODES = N_NEWS + N_ENTITY
N_USERS = 4096
N_FACTORS = 8
N_REL = 40
D = 100
Dp = 128
N_NEIGH = 20
NNZ = 262144

L = 16
NC = 2
NS = 16
NW = NC * NS
BN = 16
NBLK_NEWS = N_NEWS // BN
NBLK_ENT = N_ENTITY // BN
UPW = N_USERS // NW
CB = 128

_mesh = plsc.VectorSubcoreMesh(core_axis_name="c", subcore_axis_name="s")


def _worker_id():
    return lax.axis_index("s") * NC + lax.axis_index("c")


@functools.partial(
    pl.kernel,
    out_type=jax.ShapeDtypeStruct((N_NODES, Dp), jnp.float32),
    mesh=_mesh,
    scratch_types=[
        pltpu.VMEM((BN * N_NEIGH,), jnp.int32),
        pltpu.VMEM((BN * N_NEIGH, Dp), jnp.float32),
        pltpu.VMEM((BN * N_NEIGH, Dp), jnp.float32),
        pltpu.VMEM((BN, Dp), jnp.float32),
        pltpu.VMEM((BN, Dp), jnp.float32),
        pltpu.VMEM((N_REL, Dp), jnp.float32),
        pltpu.SemaphoreType.DMA,
    ],
)
def _node_kernel(ent_hbm, all_hbm, rel_hbm, nidx_hbm, neidx_hbm, nridx_hbm,
                 node_hbm, idx_v, gath_v, g2_v, own_v, out_v, rel_v, sem):
    w = _worker_id()
    pltpu.sync_copy(rel_hbm, rel_v)

    @pl.loop(w, NBLK_NEWS, step=NW)
    def _news_block(b):
        base = b * BN
        pltpu.sync_copy(nidx_hbm.at[pl.ds(base * N_NEIGH, BN * N_NEIGH)], idx_v)
        pltpu.async_copy(ent_hbm.at[idx_v], gath_v, sem).wait()
        pltpu.sync_copy(all_hbm.at[pl.ds(base, BN)], own_v)

        @pl.loop(0, BN)
        def _row(r):
            for ci in range(Dp // L):
                sl = pl.ds(ci * L, L)
                acc = gath_v[r * N_NEIGH, sl]
                for j in range(1, N_NEIGH):
                    acc = acc + gath_v[r * N_NEIGH + j, sl]
                out_v[r, sl] = acc * (1.0 / N_NEIGH) + rel_v[0, sl] + own_v[r, sl]

        pltpu.sync_copy(out_v, node_hbm.at[pl.ds(base, BN)])

    @pl.loop(w, NBLK_ENT, step=NW)
    def _ent_block(b):
        base = b * BN
        pltpu.sync_copy(neidx_hbm.at[pl.ds(base * N_NEIGH, BN * N_NEIGH)], idx_v)
        pltpu.async_copy(all_hbm.at[idx_v], gath_v, sem).wait()
        pltpu.sync_copy(nridx_hbm.at[pl.ds(base * N_NEIGH, BN * N_NEIGH)], idx_v)
        pltpu.async_copy(rel_hbm.at[idx_v], g2_v, sem).wait()
        pltpu.sync_copy(all_hbm.at[pl.ds(base, BN)], own_v)

        @pl.loop(0, BN)
        def _row(r):
            for ci in range(Dp // L):
                sl = pl.ds(ci * L, L)
                acc = gath_v[r * N_NEIGH, sl] + g2_v[r * N_NEIGH, sl]
                for j in range(1, N_NEIGH):
                    acc = acc + gath_v[r * N_NEIGH + j, sl] + g2_v[r * N_NEIGH + j, sl]
                out_v[r, sl] = acc * (1.0 / N_NEIGH) + own_v[r, sl]

        pltpu.sync_copy(out_v, node_hbm.at[pl.ds(N_NEWS + base, BN)])


@functools.partial(
    pl.kernel,
    out_type=jax.ShapeDtypeStruct((N_USERS, Dp), jnp.float32),
    mesh=_mesh,
    scratch_types=[
        pltpu.VMEM((48,), jnp.int32),
        pltpu.VMEM((CB,), jnp.int32),
        pltpu.VMEM((CB,), jnp.float32),
        pltpu.VMEM((CB,), jnp.int32),
        pltpu.VMEM((CB, Dp), jnp.float32),
        pltpu.VMEM((UPW * Dp,), jnp.float32),
        pltpu.VMEM((UPW, Dp), jnp.float32),
        pltpu.SemaphoreType.DMA,
    ],
)
def _user_kernel(node_hbm, cols_hbm, vals_hbm, rows_hbm, bounds_hbm, mod_hbm,
                 out_hbm, bnd_v, idx_v, val_v, row_v, gath_v, acc_v, mod_v, sem):
    w = _worker_id()
    pltpu.sync_copy(bounds_hbm, bnd_v)
    bvec = bnd_v[pl.ds(w, L)]
    start = bvec[0]
    end = bvec[1]
    astart = (start // 8) * 8
    ubase = w * UPW

    @pl.loop(0, UPW * Dp // L)
    def _zero(i):
        acc_v[pl.ds(i * L, L)] = jnp.zeros((L,), jnp.float32)

    nchunks = (end - astart + CB - 1) // CB

    @pl.loop(0, nchunks)
    def _chunk(k):
        cbase = astart + k * CB
        pltpu.sync_copy(cols_hbm.at[pl.ds(cbase, CB)], idx_v)
        pltpu.async_copy(node_hbm.at[idx_v], gath_v, sem).wait()
        pltpu.sync_copy(vals_hbm.at[pl.ds(cbase, CB)], val_v)
        pltpu.sync_copy(rows_hbm.at[pl.ds(cbase, CB)], row_v)
        nrem = end - cbase

        @pl.loop(0, CB // L)
        def _group(g):
            rvec = row_v[pl.ds(g * L, L)]
            vvec = val_v[pl.ds(g * L, L)]
            for j in range(L):
                i = g * L + j
                u = rvec[j] - ubase

                @pl.when((i < nrem) & (u >= 0) & (u < UPW))
                def _():
                    vv = vvec[j]
                    for ci in range(Dp // L):
                        plsc.addupdate(acc_v.at[pl.ds(u * Dp + ci * L, L)],
                                       gath_v[i, pl.ds(ci * L, L)] * vv)

    pltpu.sync_copy(mod_hbm.at[pl.ds(ubase, UPW)], mod_v)

    @pl.loop(0, UPW)
    def _row(r):
        for ci in range(Dp // L):
            sl = pl.ds(ci * L, L)
            mod_v[r, sl] = mod_v[r, sl] * acc_v[pl.ds(r * Dp + ci * L, L)]

    pltpu.sync_copy(mod_v, out_hbm.at[pl.ds(ubase, UPW)])


def _mod_body(ue_ref, le_ref, da_ref, wt_ref, out_ref):
    score = jax.nn.softmax(
        jnp.dot(ue_ref[...], le_ref[...].T, preferred_element_type=jnp.float32),
        axis=1)
    dw = jnp.dot(jax.nn.softmax(da_ref[...], axis=-1), wt_ref[...],
                 preferred_element_type=jnp.float32)
    out_ref[...] = 1.0 + jnp.dot(score, dw, preferred_element_type=jnp.float32)


def kernel(user_emb, all_embedding, entity_emb, relation_emb, latent_emb, weight,
           disen_weight_att, interact_vals, news_entities, news_relations,
           neigh_entities, neigh_relations, interact_rows, interact_cols):
    pad = ((0, 0), (0, Dp - D))
    ent_p = jnp.pad(entity_emb, pad)
    all_p = jnp.pad(all_embedding, pad)
    rel_p = jnp.pad(relation_emb, pad)
    wt_p = jnp.pad(weight, pad)

    nidx = news_entities.reshape(-1)
    neidx = neigh_entities.reshape(-1)
    nridx = neigh_relations.reshape(-1)

    cols_p = jnp.concatenate([interact_cols, jnp.zeros((CB,), jnp.int32)])
    vals_p = jnp.concatenate([interact_vals, jnp.zeros((CB,), jnp.float32)])
    rows_p = jnp.concatenate(
        [interact_rows, jnp.full((CB,), N_USERS - 1, jnp.int32)])
    bounds = jnp.searchsorted(
        interact_rows, jnp.arange(0, N_USERS + 1, UPW, dtype=jnp.int32)
    ).astype(jnp.int32)
    bounds = jnp.pad(bounds, (0, 48 - (NW + 1)))

    mod = pl.pallas_call(
        _mod_body,
        out_shape=jax.ShapeDtypeStruct((N_USERS, Dp), jnp.float32),
    )(user_emb, latent_emb, disen_weight_att, wt_p)

    node_p = _node_kernel(ent_p, all_p, rel_p, nidx, neidx, nridx)
    user_p = _user_kernel(node_p, cols_p, vals_p, rows_p, bounds, mod)

    return (node_p[:, :D], user_p[:, :D])


# speedup vs baseline: 7.3716x; 1.9169x over previous
import functools

import jax
import jax.numpy as jnp
from jax import lax
from jax.experimental import pallas as pl
from jax.experimental.pallas import tpu as pltpu
from jax.experimental.pallas import tpu_sc as plsc

N_NEWS = 10000
N_ENTITY = 30000
N_NODES = N_NEWS + N_ENTITY
N_USERS = 4096
N_FACTORS = 8
N_REL = 40
D = 100
Dp = 128
N_NEIGH = 20
NNZ = 262144

L = 16
NC = 2
NS = 16
NW = NC * NS
BN = 16
NBLK_NEWS = N_NEWS // BN
NBLK_ENT = N_ENTITY // BN
UPW = N_USERS // NW
CB = 128
NCH = Dp // L

_mesh = plsc.VectorSubcoreMesh(core_axis_name="c", subcore_axis_name="s")


def _worker_id():
    return lax.axis_index("s") * NC + lax.axis_index("c")


@functools.partial(
    pl.kernel,
    out_type=jax.ShapeDtypeStruct((N_NODES, Dp), jnp.float32),
    mesh=_mesh,
    scratch_types=[
        pltpu.VMEM((BN * N_NEIGH,), jnp.int32),
        pltpu.VMEM((BN * N_NEIGH,), jnp.int32),
        pltpu.VMEM((BN * N_NEIGH, Dp), jnp.float32),
        pltpu.VMEM((BN * N_NEIGH, Dp), jnp.float32),
        pltpu.VMEM((BN, Dp), jnp.float32),
        pltpu.VMEM((BN, Dp), jnp.float32),
        pltpu.SemaphoreType.DMA,
        pltpu.SemaphoreType.DMA,
    ],
)
def _node_kernel(ent_hbm, all_hbm, bias_hbm, nidx_hbm, neidx_hbm, node_hbm,
                 idx0, idx1, g0, g1, bias_v, out_v, sem0, sem1):
    w = _worker_id()

    def _phase(nblk, idx_hbm, table_hbm, row_off):
        nb = (nblk - w + NW - 1) // NW

        def issue(m, ib, gb, sem):
            base = (w + m * NW) * BN
            pltpu.sync_copy(idx_hbm.at[pl.ds(base * N_NEIGH, BN * N_NEIGH)], ib)
            pltpu.async_copy(table_hbm.at[ib], gb, sem)

        def finish(m, ib, gb, sem):
            base = (w + m * NW) * BN
            pltpu.make_async_copy(table_hbm.at[ib], gb, sem).wait()
            pltpu.sync_copy(bias_hbm.at[pl.ds(row_off + base, BN)], bias_v)

            @pl.loop(0, BN)
            def _row(r):
                for ci in range(NCH):
                    sl = pl.ds(ci * L, L)
                    acc = gb[r * N_NEIGH, sl]
                    for j in range(1, N_NEIGH):
                        acc = acc + gb[r * N_NEIGH + j, sl]
                    out_v[r, sl] = acc * (1.0 / N_NEIGH) + bias_v[r, sl]

            pltpu.sync_copy(out_v, node_hbm.at[pl.ds(row_off + base, BN)])

        issue(0, idx0, g0, sem0)

        @pl.loop(0, (nb + 1) // 2)
        def _pair(p):
            m0 = 2 * p
            m1 = m0 + 1

            @pl.when(m1 < nb)
            def _():
                issue(m1, idx1, g1, sem1)

            finish(m0, idx0, g0, sem0)

            @pl.when(m0 + 2 < nb)
            def _():
                issue(m0 + 2, idx0, g0, sem0)

            @pl.when(m1 < nb)
            def _():
                finish(m1, idx1, g1, sem1)

    _phase(NBLK_NEWS, nidx_hbm, ent_hbm, 0)
    _phase(NBLK_ENT, neidx_hbm, all_hbm, N_NEWS)


@functools.partial(
    pl.kernel,
    out_type=jax.ShapeDtypeStruct((N_USERS, Dp), jnp.float32),
    mesh=_mesh,
    scratch_types=[
        pltpu.VMEM((48,), jnp.int32),
        pltpu.VMEM((CB,), jnp.int32),
        pltpu.VMEM((CB,), jnp.int32),
        pltpu.VMEM((CB,), jnp.float32),
        pltpu.VMEM((CB,), jnp.int32),
        pltpu.VMEM((CB, Dp), jnp.float32),
        pltpu.VMEM((CB, Dp), jnp.float32),
        pltpu.VMEM((UPW * Dp,), jnp.float32),
        pltpu.VMEM((UPW, Dp), jnp.float32),
        pltpu.SemaphoreType.DMA,
        pltpu.SemaphoreType.DMA,
    ],
)
def _user_kernel(node_hbm, cols_hbm, vals_hbm, rows_hbm, bounds_hbm, mod_hbm,
                 out_hbm, bnd_v, idx0, idx1, val_v, row_v, g0, g1, acc_v,
                 mod_v, sem0, sem1):
    w = _worker_id()
    pltpu.sync_copy(bounds_hbm, bnd_v)
    bvec = bnd_v[pl.ds(w, L)]
    start = bvec[0]
    end = bvec[1]
    astart = (start // 8) * 8
    ubase = w * UPW

    @pl.loop(0, UPW * Dp // L)
    def _zero(i):
        acc_v[pl.ds(i * L, L)] = jnp.zeros((L,), jnp.float32)

    nchunks = (end - astart + CB - 1) // CB

    def issue(k, ib, gb, sem):
        cbase = astart + k * CB
        pltpu.sync_copy(cols_hbm.at[pl.ds(cbase, CB)], ib)
        pltpu.async_copy(node_hbm.at[ib], gb, sem)

    def finish(k, ib, gb, sem):
        cbase = astart + k * CB
        pltpu.make_async_copy(node_hbm.at[ib], gb, sem).wait()
        pltpu.sync_copy(vals_hbm.at[pl.ds(cbase, CB)], val_v)
        pltpu.sync_copy(rows_hbm.at[pl.ds(cbase, CB)], row_v)
        nrem = end - cbase

        @pl.loop(0, CB // L)
        def _group(g):
            rvec = row_v[pl.ds(g * L, L)]
            vvec = val_v[pl.ds(g * L, L)]
            for j in range(L):
                i = g * L + j
                u = rvec[j] - ubase

                @pl.when((i < nrem) & (u >= 0) & (u < UPW))
                def _():
                    vv = vvec[j]
                    for ci in range(NCH):
                        plsc.addupdate(acc_v.at[pl.ds(u * Dp + ci * L, L)],
                                       gb[i, pl.ds(ci * L, L)] * vv)

    @pl.when(nchunks > 0)
    def _():
        issue(0, idx0, g0, sem0)

    @pl.loop(0, (nchunks + 1) // 2)
    def _pair(p):
        k0 = 2 * p
        k1 = k0 + 1

        @pl.when(k1 < nchunks)
        def _():
            issue(k1, idx1, g1, sem1)

        finish(k0, idx0, g0, sem0)

        @pl.when(k0 + 2 < nchunks)
        def _():
            issue(k0 + 2, idx0, g0, sem0)

        @pl.when(k1 < nchunks)
        def _():
            finish(k1, idx1, g1, sem1)

    pltpu.sync_copy(mod_hbm.at[pl.ds(ubase, UPW)], mod_v)

    @pl.loop(0, UPW)
    def _row(r):
        for ci in range(NCH):
            sl = pl.ds(ci * L, L)
            mod_v[r, sl] = mod_v[r, sl] * acc_v[pl.ds(r * Dp + ci * L, L)]

    pltpu.sync_copy(mod_v, out_hbm.at[pl.ds(ubase, UPW)])


def _mod_body(ue_ref, le_ref, da_ref, wt_ref, out_ref):
    score = jax.nn.softmax(
        jnp.dot(ue_ref[...], le_ref[...].T, preferred_element_type=jnp.float32),
        axis=1)
    dw = jnp.dot(jax.nn.softmax(da_ref[...], axis=-1), wt_ref[...],
                 preferred_element_type=jnp.float32)
    out_ref[...] = 1.0 + jnp.dot(score, dw, preferred_element_type=jnp.float32)


_PB = 1000


def _pad_body(a_ref, b_ref, ap_ref, bp_ref):
    z = jnp.zeros((_PB, Dp - D), jnp.float32)
    ap_ref[...] = jnp.concatenate([a_ref[...], z], axis=1)
    bp_ref[...] = jnp.concatenate([b_ref[...], z], axis=1)


def _bias_body(all_ref, rel_ref, nr_ref, out_ref):
    i = pl.program_id(0)
    a = all_ref[...]
    rel = rel_ref[...]
    z = jnp.zeros((_PB, Dp - D), jnp.float32)

    @pl.when(i < N_NEWS // _PB)
    def _():
        out_ref[...] = jnp.concatenate([a + rel[0][None, :], z], axis=1)

    @pl.when(i >= N_NEWS // _PB)
    def _():
        nr = nr_ref[...]
        onehot = (nr[:, :, None] ==
                  lax.broadcasted_iota(jnp.int32, (_PB, N_NEIGH, N_REL), 2))
        cnt = jnp.sum(onehot.astype(jnp.float32), axis=1)
        relpart = jnp.dot(cnt, rel, preferred_element_type=jnp.float32)
        out_ref[...] = jnp.concatenate([a + relpart * (1.0 / N_NEIGH), z],
                                       axis=1)


def kernel(user_emb, all_embedding, entity_emb, relation_emb, latent_emb, weight,
           disen_weight_att, interact_vals, news_entities, news_relations,
           neigh_entities, neigh_relations, interact_rows, interact_cols):
    nidx = news_entities.reshape(-1)
    neidx = neigh_entities.reshape(-1)

    cols_p = jnp.concatenate([interact_cols, jnp.zeros((CB,), jnp.int32)])
    vals_p = jnp.concatenate([interact_vals, jnp.zeros((CB,), jnp.float32)])
    rows_p = jnp.concatenate(
        [interact_rows, jnp.full((CB,), N_USERS - 1, jnp.int32)])
    bounds = jnp.searchsorted(
        interact_rows, jnp.arange(0, N_USERS + 1, UPW, dtype=jnp.int32)
    ).astype(jnp.int32)
    bounds = jnp.pad(bounds, (0, 48 - (NW + 1)))

    wt_p = jnp.pad(weight, ((0, 0), (0, Dp - D)))

    mod = pl.pallas_call(
        _mod_body,
        out_shape=jax.ShapeDtypeStruct((N_USERS, Dp), jnp.float32),
    )(user_emb, latent_emb, disen_weight_att, wt_p)

    ent_p, all_p = pl.pallas_call(
        _pad_body,
        grid=(N_ENTITY // _PB,),
        in_specs=[pl.BlockSpec((_PB, D), lambda i: (i, 0)),
                  pl.BlockSpec((_PB, D), lambda i: (i, 0))],
        out_specs=[pl.BlockSpec((_PB, Dp), lambda i: (i, 0)),
                   pl.BlockSpec((_PB, Dp), lambda i: (i, 0))],
        out_shape=[jax.ShapeDtypeStruct((N_ENTITY, Dp), jnp.float32),
                   jax.ShapeDtypeStruct((N_ENTITY, Dp), jnp.float32)],
    )(entity_emb, all_embedding)

    nnews_blk = N_NEWS // _PB
    bias = pl.pallas_call(
        _bias_body,
        grid=(N_NODES // _PB,),
        in_specs=[
            pl.BlockSpec((_PB, D),
                         lambda i: (jnp.where(i < nnews_blk, i, i - nnews_blk), 0)),
            pl.BlockSpec((N_REL, D), lambda i: (0, 0)),
            pl.BlockSpec((_PB, N_NEIGH),
                         lambda i: (jnp.where(i < nnews_blk, 0, i - nnews_blk), 0)),
        ],
        out_specs=pl.BlockSpec((_PB, Dp), lambda i: (i, 0)),
        out_shape=jax.ShapeDtypeStruct((N_NODES, Dp), jnp.float32),
    )(all_embedding, relation_emb, neigh_relations)

    node_p = _node_kernel(ent_p, all_p, bias, nidx, neidx)
    user_p = _user_kernel(node_p, cols_p, vals_p, rows_p, bounds, mod)

    return (node_p[:, :D], user_p[:, :D])


# speedup vs baseline: 9.6679x; 1.3115x over previous
import functools

import jax
import jax.numpy as jnp
from jax import lax
from jax.experimental import pallas as pl
from jax.experimental.pallas import tpu as pltpu
from jax.experimental.pallas import tpu_sc as plsc

N_NEWS = 10000
N_ENTITY = 30000
N_NODES = N_NEWS + N_ENTITY
N_USERS = 4096
N_FACTORS = 8
N_REL = 40
D = 100
Dp = 128
N_NEIGH = 20
NNZ = 262144

L = 16
NC = 2
NS = 16
NW = NC * NS
BN = 80
NBLK_NEWS = N_NEWS // BN
NBLK_ENT = N_ENTITY // BN
UPW = N_USERS // NW
CB = 128
NCH = Dp // L

_mesh = plsc.VectorSubcoreMesh(core_axis_name="c", subcore_axis_name="s")


def _worker_id():
    return lax.axis_index("s") * NC + lax.axis_index("c")


@functools.partial(
    pl.kernel,
    out_type=jax.ShapeDtypeStruct((N_NODES, Dp), jnp.float32),
    mesh=_mesh,
    scratch_types=[
        pltpu.VMEM((N_NEIGH * BN,), jnp.int32),
        pltpu.VMEM((N_NEIGH * BN,), jnp.int32),
        pltpu.VMEM((BN, Dp), jnp.float32),
        pltpu.VMEM((BN, Dp), jnp.float32),
        pltpu.VMEM((BN, Dp), jnp.float32),
        pltpu.SemaphoreType.DMA,
        pltpu.SemaphoreType.DMA,
    ],
)
def _node_kernel(ent_hbm, all_hbm, bias_hbm, nidx_hbm, neidx_hbm, node_hbm,
                 idx0, idx1, g0, g1, bias_v, sem0, sem1):
    w = _worker_id()

    def _phase(nblk, idx_hbm, table_hbm, row_off):
        nb = (nblk - w + NW - 1) // NW

        def issue(m, ib, gb, sem):
            base = (w + m * NW) * BN
            @pl.loop(0, BN)
            def _z(r):
                for ci in range(NCH):
                    gb[r, pl.ds(ci * L, L)] = jnp.zeros((L,), jnp.float32)

            b = w + m * NW
            pltpu.sync_copy(
                idx_hbm.at[pl.ds(b * N_NEIGH * BN, N_NEIGH * BN)], ib)
            for j in range(N_NEIGH):
                pltpu.async_copy(table_hbm.at[ib.at[pl.ds(j * BN, BN)]], gb,
                                 sem, add=True)

        def finish(m, ib, gb, sem):
            base = (w + m * NW) * BN
            for j in range(N_NEIGH):
                pltpu.make_async_copy(
                    table_hbm.at[ib.at[pl.ds(j * BN, BN)]], gb, sem).wait()
            pltpu.sync_copy(bias_hbm.at[pl.ds(row_off + base, BN)], bias_v)

            @pl.loop(0, BN)
            def _row(r):
                for ci in range(NCH):
                    sl = pl.ds(ci * L, L)
                    gb[r, sl] = gb[r, sl] * (1.0 / N_NEIGH) + bias_v[r, sl]

            pltpu.sync_copy(gb, node_hbm.at[pl.ds(row_off + base, BN)])

        issue(0, idx0, g0, sem0)

        @pl.loop(0, (nb + 1) // 2)
        def _pair(p):
            m0 = 2 * p
            m1 = m0 + 1

            @pl.when(m1 < nb)
            def _():
                issue(m1, idx1, g1, sem1)

            finish(m0, idx0, g0, sem0)

            @pl.when(m0 + 2 < nb)
            def _():
                issue(m0 + 2, idx0, g0, sem0)

            @pl.when(m1 < nb)
            def _():
                finish(m1, idx1, g1, sem1)

    _phase(NBLK_NEWS, nidx_hbm, ent_hbm, 0)
    _phase(NBLK_ENT, neidx_hbm, all_hbm, N_NEWS)


@functools.partial(
    pl.kernel,
    out_type=jax.ShapeDtypeStruct((N_USERS, Dp), jnp.float32),
    mesh=_mesh,
    scratch_types=[
        pltpu.VMEM((48,), jnp.int32),
        pltpu.VMEM((CB,), jnp.int32),
        pltpu.VMEM((CB,), jnp.int32),
        pltpu.VMEM((CB,), jnp.float32),
        pltpu.VMEM((CB,), jnp.int32),
        pltpu.VMEM((CB, Dp), jnp.float32),
        pltpu.VMEM((CB, Dp), jnp.float32),
        pltpu.VMEM((UPW * Dp,), jnp.float32),
        pltpu.VMEM((UPW, Dp), jnp.float32),
        pltpu.SemaphoreType.DMA,
        pltpu.SemaphoreType.DMA,
    ],
)
def _user_kernel(node_hbm, cols_hbm, vals_hbm, rows_hbm, bounds_hbm, mod_hbm,
                 out_hbm, bnd_v, idx0, idx1, val_v, row_v, g0, g1, acc_v,
                 mod_v, sem0, sem1):
    w = _worker_id()
    pltpu.sync_copy(bounds_hbm, bnd_v)
    bvec = bnd_v[pl.ds(w, L)]
    start = bvec[0]
    end = bvec[1]
    astart = (start // 8) * 8
    ubase = w * UPW

    @pl.loop(0, UPW * Dp // L)
    def _zero(i):
        acc_v[pl.ds(i * L, L)] = jnp.zeros((L,), jnp.float32)

    nchunks = (end - astart + CB - 1) // CB

    def issue(k, ib, gb, sem):
        cbase = astart + k * CB
        pltpu.sync_copy(cols_hbm.at[pl.ds(cbase, CB)], ib)
        pltpu.async_copy(node_hbm.at[ib], gb, sem)

    def finish(k, ib, gb, sem):
        cbase = astart + k * CB
        pltpu.make_async_copy(node_hbm.at[ib], gb, sem).wait()
        pltpu.sync_copy(vals_hbm.at[pl.ds(cbase, CB)], val_v)
        pltpu.sync_copy(rows_hbm.at[pl.ds(cbase, CB)], row_v)
        nrem = end - cbase

        @pl.loop(0, CB // L)
        def _group(g):
            rvec = row_v[pl.ds(g * L, L)]
            vvec = val_v[pl.ds(g * L, L)]
            for j in range(L):
                i = g * L + j
                u = rvec[j] - ubase

                @pl.when((i < nrem) & (u >= 0) & (u < UPW))
                def _():
                    vv = vvec[j]
                    for ci in range(NCH):
                        plsc.addupdate(acc_v.at[pl.ds(u * Dp + ci * L, L)],
                                       gb[i, pl.ds(ci * L, L)] * vv)

    @pl.when(nchunks > 0)
    def _():
        issue(0, idx0, g0, sem0)

    @pl.loop(0, (nchunks + 1) // 2)
    def _pair(p):
        k0 = 2 * p
        k1 = k0 + 1

        @pl.when(k1 < nchunks)
        def _():
            issue(k1, idx1, g1, sem1)

        finish(k0, idx0, g0, sem0)

        @pl.when(k0 + 2 < nchunks)
        def _():
            issue(k0 + 2, idx0, g0, sem0)

        @pl.when(k1 < nchunks)
        def _():
            finish(k1, idx1, g1, sem1)

    pltpu.sync_copy(mod_hbm.at[pl.ds(ubase, UPW)], mod_v)

    @pl.loop(0, UPW)
    def _row(r):
        for ci in range(NCH):
            sl = pl.ds(ci * L, L)
            mod_v[r, sl] = mod_v[r, sl] * acc_v[pl.ds(r * Dp + ci * L, L)]

    pltpu.sync_copy(mod_v, out_hbm.at[pl.ds(ubase, UPW)])


def _mod_body(ue_ref, le_ref, da_ref, wt_ref, out_ref):
    score = jax.nn.softmax(
        jnp.dot(ue_ref[...], le_ref[...].T, preferred_element_type=jnp.float32),
        axis=1)
    dw = jnp.dot(jax.nn.softmax(da_ref[...], axis=-1), wt_ref[...],
                 preferred_element_type=jnp.float32)
    out_ref[...] = 1.0 + jnp.dot(score, dw, preferred_element_type=jnp.float32)


_PB = 1000


def _pad_body(a_ref, b_ref, ap_ref, bp_ref):
    z = jnp.zeros((_PB, Dp - D), jnp.float32)
    ap_ref[...] = jnp.concatenate([a_ref[...], z], axis=1)
    bp_ref[...] = jnp.concatenate([b_ref[...], z], axis=1)


def _bias_body(all_ref, rel_ref, nr_ref, out_ref):
    i = pl.program_id(0)
    a = all_ref[...]
    rel = rel_ref[...]
    z = jnp.zeros((_PB, Dp - D), jnp.float32)

    @pl.when(i < N_NEWS // _PB)
    def _():
        out_ref[...] = jnp.concatenate([a + rel[0][None, :], z], axis=1)

    @pl.when(i >= N_NEWS // _PB)
    def _():
        nr = nr_ref[...]
        onehot = (nr[:, :, None] ==
                  lax.broadcasted_iota(jnp.int32, (_PB, N_NEIGH, N_REL), 2))
        cnt = jnp.sum(onehot.astype(jnp.float32), axis=1)
        relpart = jnp.dot(cnt, rel, preferred_element_type=jnp.float32)
        out_ref[...] = jnp.concatenate([a + relpart * (1.0 / N_NEIGH), z],
                                       axis=1)


def kernel(user_emb, all_embedding, entity_emb, relation_emb, latent_emb, weight,
           disen_weight_att, interact_vals, news_entities, news_relations,
           neigh_entities, neigh_relations, interact_rows, interact_cols):
    nidx = news_entities.reshape(NBLK_NEWS, BN, N_NEIGH).transpose(0, 2, 1).reshape(-1)
    neidx = neigh_entities.reshape(NBLK_ENT, BN, N_NEIGH).transpose(0, 2, 1).reshape(-1)

    cols_p = jnp.concatenate([interact_cols, jnp.zeros((CB,), jnp.int32)])
    vals_p = jnp.concatenate([interact_vals, jnp.zeros((CB,), jnp.float32)])
    rows_p = jnp.concatenate(
        [interact_rows, jnp.full((CB,), N_USERS - 1, jnp.int32)])
    bounds = jnp.searchsorted(
        interact_rows, jnp.arange(0, N_USERS + 1, UPW, dtype=jnp.int32)
    ).astype(jnp.int32)
    bounds = jnp.pad(bounds, (0, 48 - (NW + 1)))

    wt_p = jnp.pad(weight, ((0, 0), (0, Dp - D)))

    mod = pl.pallas_call(
        _mod_body,
        out_shape=jax.ShapeDtypeStruct((N_USERS, Dp), jnp.float32),
    )(user_emb, latent_emb, disen_weight_att, wt_p)

    ent_p, all_p = pl.pallas_call(
        _pad_body,
        grid=(N_ENTITY // _PB,),
        in_specs=[pl.BlockSpec((_PB, D), lambda i: (i, 0)),
                  pl.BlockSpec((_PB, D), lambda i: (i, 0))],
        out_specs=[pl.BlockSpec((_PB, Dp), lambda i: (i, 0)),
                   pl.BlockSpec((_PB, Dp), lambda i: (i, 0))],
        out_shape=[jax.ShapeDtypeStruct((N_ENTITY, Dp), jnp.float32),
                   jax.ShapeDtypeStruct((N_ENTITY, Dp), jnp.float32)],
    )(entity_emb, all_embedding)

    nnews_blk = N_NEWS // _PB
    bias = pl.pallas_call(
        _bias_body,
        grid=(N_NODES // _PB,),
        in_specs=[
            pl.BlockSpec((_PB, D),
                         lambda i: (jnp.where(i < nnews_blk, i, i - nnews_blk), 0)),
            pl.BlockSpec((N_REL, D), lambda i: (0, 0)),
            pl.BlockSpec((_PB, N_NEIGH),
                         lambda i: (jnp.where(i < nnews_blk, 0, i - nnews_blk), 0)),
        ],
        out_specs=pl.BlockSpec((_PB, Dp), lambda i: (i, 0)),
        out_shape=jax.ShapeDtypeStruct((N_NODES, Dp), jnp.float32),
    )(all_embedding, relation_emb, neigh_relations)

    node_p = _node_kernel(ent_p, all_p, bias, nidx, neidx)
    user_p = _user_kernel(node_p, cols_p, vals_p, rows_p, bounds, mod)

    return (node_p[:, :D], user_p[:, :D])


# speedup vs baseline: 13.2948x; 1.3751x over previous
import functools

import jax
import jax.numpy as jnp
from jax import lax
from jax.experimental import pallas as pl
from jax.experimental.pallas import tpu as pltpu
from jax.experimental.pallas import tpu_sc as plsc

N_NEWS = 10000
N_ENTITY = 30000
N_NODES = N_NEWS + N_ENTITY
N_USERS = 4096
N_FACTORS = 8
N_REL = 40
D = 100
Dp = 128
N_NEIGH = 20
NNZ = 262144

L = 16
NC = 2
NS = 16
NW = NC * NS
BN = 80
NBLK_NEWS = N_NEWS // BN
NBLK_ENT = N_ENTITY // BN
UPW = N_USERS // NW
CB = 128
NCH = Dp // L

_mesh = plsc.VectorSubcoreMesh(core_axis_name="c", subcore_axis_name="s")


def _worker_id():
    return lax.axis_index("s") * NC + lax.axis_index("c")


@functools.partial(
    pl.kernel,
    out_type=jax.ShapeDtypeStruct((N_NODES, Dp), jnp.float32),
    mesh=_mesh,
    scratch_types=[
        pltpu.VMEM((N_NEIGH * BN,), jnp.int32),
        pltpu.VMEM((N_NEIGH * BN,), jnp.int32),
        pltpu.VMEM((BN, Dp), jnp.float32),
        pltpu.VMEM((BN, Dp), jnp.float32),
        pltpu.VMEM((BN, Dp), jnp.float32),
        pltpu.SemaphoreType.DMA,
        pltpu.SemaphoreType.DMA,
    ],
)
def _node_kernel(ent_hbm, all_hbm, bias_hbm, nidx_hbm, neidx_hbm, node_hbm,
                 idx0, idx1, g0, g1, bias_v, sem0, sem1):
    w = _worker_id()

    def _phase(nblk, idx_hbm, table_hbm, row_off):
        nb = (nblk - w + NW - 1) // NW

        def issue(m, ib, gb, sem):
            base = (w + m * NW) * BN
            @pl.loop(0, BN)
            def _z(r):
                for ci in range(NCH):
                    gb[r, pl.ds(ci * L, L)] = jnp.zeros((L,), jnp.float32)

            b = w + m * NW
            pltpu.sync_copy(
                idx_hbm.at[pl.ds(b * N_NEIGH * BN, N_NEIGH * BN)], ib)
            for j in range(N_NEIGH):
                pltpu.async_copy(table_hbm.at[ib.at[pl.ds(j * BN, BN)]], gb,
                                 sem, add=True)

        def finish(m, ib, gb, sem):
            base = (w + m * NW) * BN
            for j in range(N_NEIGH):
                pltpu.make_async_copy(
                    table_hbm.at[ib.at[pl.ds(j * BN, BN)]], gb, sem).wait()
            pltpu.sync_copy(bias_hbm.at[pl.ds(row_off + base, BN)], bias_v)

            @pl.loop(0, BN)
            def _row(r):
                for ci in range(NCH):
                    sl = pl.ds(ci * L, L)
                    gb[r, sl] = gb[r, sl] * (1.0 / N_NEIGH) + bias_v[r, sl]

            pltpu.sync_copy(gb, node_hbm.at[pl.ds(row_off + base, BN)])

        issue(0, idx0, g0, sem0)

        @pl.loop(0, (nb + 1) // 2)
        def _pair(p):
            m0 = 2 * p
            m1 = m0 + 1

            @pl.when(m1 < nb)
            def _():
                issue(m1, idx1, g1, sem1)

            finish(m0, idx0, g0, sem0)

            @pl.when(m0 + 2 < nb)
            def _():
                issue(m0 + 2, idx0, g0, sem0)

            @pl.when(m1 < nb)
            def _():
                finish(m1, idx1, g1, sem1)

    _phase(NBLK_NEWS, nidx_hbm, ent_hbm, 0)
    _phase(NBLK_ENT, neidx_hbm, all_hbm, N_NEWS)


NNZ_PER_TILE = NNZ // NW
NCHUNK = NNZ_PER_TILE // CB
URT = N_USERS // NS


@functools.partial(
    pl.kernel,
    out_type=jax.ShapeDtypeStruct((NC, N_USERS, Dp), jnp.float32),
    mesh=_mesh,
    scratch_types=[
        pltpu.VMEM((CB,), jnp.int32),
        pltpu.VMEM((CB,), jnp.int32),
        pltpu.VMEM((CB,), jnp.int32),
        pltpu.VMEM((CB,), jnp.int32),
        pltpu.VMEM((CB,), jnp.float32),
        pltpu.VMEM((CB, Dp), jnp.float32),
        pltpu.VMEM((CB, Dp), jnp.float32),
        pltpu.VMEM_SHARED((N_USERS, Dp), jnp.float32),
        pltpu.SemaphoreType.DMA,
        pltpu.SemaphoreType.DMA,
        pltpu.SemaphoreType.DMA,
        pltpu.SemaphoreType.DMA,
    ],
)
def _user_kernel(node_hbm, cols_hbm, vals_hbm, rows_hbm, out_hbm,
                 idx0, idx1, row0, row1, val_v, g0, g1, acc_sh,
                 semg0, semg1, sems0, sems1):
    c = lax.axis_index("c")
    s = lax.axis_index("s")
    gid = s * NC + c
    base = gid * NNZ_PER_TILE

    @pl.loop(0, CB)
    def _z(r):
        for ci in range(NCH):
            g0[r, pl.ds(ci * L, L)] = jnp.zeros((L,), jnp.float32)

    pltpu.sync_copy(g0, acc_sh.at[pl.ds(s * URT, CB)])
    pltpu.sync_copy(g0, acc_sh.at[pl.ds(s * URT + CB, CB)])
    plsc.subcore_barrier()

    def issue(k, ib, rb, gb, semg, sems, drain):
        if drain:
            pltpu.make_async_copy(node_hbm.at[pl.ds(0, CB)], gb, sems).wait()
        cbase = base + k * CB
        pltpu.sync_copy(cols_hbm.at[pl.ds(cbase, CB)], ib)
        pltpu.sync_copy(rows_hbm.at[pl.ds(cbase, CB)], rb)
        pltpu.async_copy(node_hbm.at[ib], gb, semg)

    def finish(k, ib, rb, gb, semg, sems):
        cbase = base + k * CB
        pltpu.make_async_copy(node_hbm.at[ib], gb, semg).wait()
        pltpu.sync_copy(vals_hbm.at[pl.ds(cbase, CB)], val_v)

        @pl.loop(0, CB // L)
        def _group(g):
            vvec = val_v[pl.ds(g * L, L)]
            for j in range(L):
                r = g * L + j
                vv = vvec[j]
                for ci in range(NCH):
                    sl = pl.ds(ci * L, L)
                    gb[r, sl] = gb[r, sl] * vv

        pltpu.async_copy(gb, acc_sh.at[rb], sems, add=True)

    issue(0, idx0, row0, g0, semg0, sems0, drain=False)
    issue(1, idx1, row1, g1, semg1, sems1, drain=False)

    @pl.loop(0, NCHUNK // 2)
    def _pair(p):
        k0 = 2 * p
        k1 = k0 + 1
        finish(k0, idx0, row0, g0, semg0, sems0)

        @pl.when(k0 + 2 < NCHUNK)
        def _():
            issue(k0 + 2, idx0, row0, g0, semg0, sems0, drain=True)

        finish(k1, idx1, row1, g1, semg1, sems1)

        @pl.when(k1 + 2 < NCHUNK)
        def _():
            issue(k1 + 2, idx1, row1, g1, semg1, sems1, drain=True)

    pltpu.make_async_copy(node_hbm.at[pl.ds(0, CB)], g0, sems0).wait()
    pltpu.make_async_copy(node_hbm.at[pl.ds(0, CB)], g1, sems1).wait()
    plsc.subcore_barrier()
    pltpu.sync_copy(acc_sh.at[pl.ds(s * URT, URT)],
                    out_hbm.at[c].at[pl.ds(s * URT, URT)])


@functools.partial(
    pl.kernel,
    out_type=jax.ShapeDtypeStruct((N_USERS, Dp), jnp.float32),
    mesh=_mesh,
    scratch_types=[
        pltpu.VMEM((UPW, Dp), jnp.float32),
        pltpu.VMEM((UPW, Dp), jnp.float32),
        pltpu.VMEM((UPW, Dp), jnp.float32),
    ],
)
def _combine_kernel(part_hbm, mod_hbm, out_hbm, a_v, b_v, m_v):
    w = _worker_id()
    ubase = w * UPW
    pltpu.sync_copy(part_hbm.at[0].at[pl.ds(ubase, UPW)], a_v)
    pltpu.sync_copy(part_hbm.at[1].at[pl.ds(ubase, UPW)], b_v)
    pltpu.sync_copy(mod_hbm.at[pl.ds(ubase, UPW)], m_v)

    @pl.loop(0, UPW)
    def _row(r):
        for ci in range(NCH):
            sl = pl.ds(ci * L, L)
            m_v[r, sl] = (a_v[r, sl] + b_v[r, sl]) * m_v[r, sl]

    pltpu.sync_copy(m_v, out_hbm.at[pl.ds(ubase, UPW)])


def _mod_body(ue_ref, le_ref, da_ref, wt_ref, out_ref):
    score = jax.nn.softmax(
        jnp.dot(ue_ref[...], le_ref[...].T, preferred_element_type=jnp.float32),
        axis=1)
    dw = jnp.dot(jax.nn.softmax(da_ref[...], axis=-1), wt_ref[...],
                 preferred_element_type=jnp.float32)
    out_ref[...] = 1.0 + jnp.dot(score, dw, preferred_element_type=jnp.float32)


_PB = 1000


def _pad_body(a_ref, b_ref, ap_ref, bp_ref):
    z = jnp.zeros((_PB, Dp - D), jnp.float32)
    ap_ref[...] = jnp.concatenate([a_ref[...], z], axis=1)
    bp_ref[...] = jnp.concatenate([b_ref[...], z], axis=1)


def _bias_body(all_ref, rel_ref, nr_ref, out_ref):
    i = pl.program_id(0)
    a = all_ref[...]
    rel = rel_ref[...]
    z = jnp.zeros((_PB, Dp - D), jnp.float32)

    @pl.when(i < N_NEWS // _PB)
    def _():
        out_ref[...] = jnp.concatenate([a + rel[0][None, :], z], axis=1)

    @pl.when(i >= N_NEWS // _PB)
    def _():
        nr = nr_ref[...]
        onehot = (nr[:, :, None] ==
                  lax.broadcasted_iota(jnp.int32, (_PB, N_NEIGH, N_REL), 2))
        cnt = jnp.sum(onehot.astype(jnp.float32), axis=1)
        relpart = jnp.dot(cnt, rel, preferred_element_type=jnp.float32)
        out_ref[...] = jnp.concatenate([a + relpart * (1.0 / N_NEIGH), z],
                                       axis=1)


def kernel(user_emb, all_embedding, entity_emb, relation_emb, latent_emb, weight,
           disen_weight_att, interact_vals, news_entities, news_relations,
           neigh_entities, neigh_relations, interact_rows, interact_cols):
    nidx = news_entities.reshape(NBLK_NEWS, BN, N_NEIGH).transpose(0, 2, 1).reshape(-1)
    neidx = neigh_entities.reshape(NBLK_ENT, BN, N_NEIGH).transpose(0, 2, 1).reshape(-1)

    wt_p = jnp.pad(weight, ((0, 0), (0, Dp - D)))

    mod = pl.pallas_call(
        _mod_body,
        out_shape=jax.ShapeDtypeStruct((N_USERS, Dp), jnp.float32),
    )(user_emb, latent_emb, disen_weight_att, wt_p)

    ent_p, all_p = pl.pallas_call(
        _pad_body,
        grid=(N_ENTITY // _PB,),
        in_specs=[pl.BlockSpec((_PB, D), lambda i: (i, 0)),
                  pl.BlockSpec((_PB, D), lambda i: (i, 0))],
        out_specs=[pl.BlockSpec((_PB, Dp), lambda i: (i, 0)),
                   pl.BlockSpec((_PB, Dp), lambda i: (i, 0))],
        out_shape=[jax.ShapeDtypeStruct((N_ENTITY, Dp), jnp.float32),
                   jax.ShapeDtypeStruct((N_ENTITY, Dp), jnp.float32)],
    )(entity_emb, all_embedding)

    nnews_blk = N_NEWS // _PB
    bias = pl.pallas_call(
        _bias_body,
        grid=(N_NODES // _PB,),
        in_specs=[
            pl.BlockSpec((_PB, D),
                         lambda i: (jnp.where(i < nnews_blk, i, i - nnews_blk), 0)),
            pl.BlockSpec((N_REL, D), lambda i: (0, 0)),
            pl.BlockSpec((_PB, N_NEIGH),
                         lambda i: (jnp.where(i < nnews_blk, 0, i - nnews_blk), 0)),
        ],
        out_specs=pl.BlockSpec((_PB, Dp), lambda i: (i, 0)),
        out_shape=jax.ShapeDtypeStruct((N_NODES, Dp), jnp.float32),
    )(all_embedding, relation_emb, neigh_relations)

    node_p = _node_kernel(ent_p, all_p, bias, nidx, neidx)
    part = _user_kernel(node_p, interact_cols, interact_vals, interact_rows)
    user_p = _combine_kernel(part, mod)

    return (node_p[:, :D], user_p[:, :D])


# speedup vs baseline: 15.6010x; 1.1735x over previous
import functools

import jax
import jax.numpy as jnp
from jax import lax
from jax.experimental import pallas as pl
from jax.experimental.pallas import tpu as pltpu
from jax.experimental.pallas import tpu_sc as plsc

N_NEWS = 10000
N_ENTITY = 30000
N_NODES = N_NEWS + N_ENTITY
N_USERS = 4096
N_FACTORS = 8
N_REL = 40
D = 100
Dp = 128
N_NEIGH = 20
NNZ = 262144

L = 16
NC = 2
NS = 16
NW = NC * NS
BN = 80
NBLK_NEWS = N_NEWS // BN
NBLK_ENT = N_ENTITY // BN
UPW = N_USERS // NW
CB = 128
NCH = Dp // L

_mesh = plsc.VectorSubcoreMesh(core_axis_name="c", subcore_axis_name="s")


def _worker_id():
    return lax.axis_index("s") * NC + lax.axis_index("c")


@functools.partial(
    pl.kernel,
    out_type=jax.ShapeDtypeStruct((N_NODES, Dp), jnp.float32),
    mesh=_mesh,
    scratch_types=[
        pltpu.VMEM((N_NEIGH * BN,), jnp.int32),
        pltpu.VMEM((N_NEIGH * BN,), jnp.int32),
        pltpu.VMEM((BN, Dp), jnp.float32),
        pltpu.VMEM((BN, Dp), jnp.float32),
        pltpu.SemaphoreType.DMA,
        pltpu.SemaphoreType.DMA,
    ],
)
def _node_kernel(ent_hbm, all_hbm, nidx_hbm, neidx_hbm, agg_hbm,
                 idx0, idx1, g0, g1, sem0, sem1):
    w = _worker_id()

    def _phase(nblk, idx_hbm, table_hbm, row_off):
        nb = (nblk - w + NW - 1) // NW

        def issue(m, ib, gb, sem):
            @pl.loop(0, BN)
            def _z(r):
                for ci in range(NCH):
                    gb[r, pl.ds(ci * L, L)] = jnp.zeros((L,), jnp.float32)

            b = w + m * NW
            pltpu.sync_copy(
                idx_hbm.at[pl.ds(b * N_NEIGH * BN, N_NEIGH * BN)], ib)
            for j in range(N_NEIGH):
                pltpu.async_copy(table_hbm.at[ib.at[pl.ds(j * BN, BN)]], gb,
                                 sem, add=True)

        def finish(m, ib, gb, sem):
            base = (w + m * NW) * BN
            for j in range(N_NEIGH):
                pltpu.make_async_copy(
                    table_hbm.at[ib.at[pl.ds(j * BN, BN)]], gb, sem).wait()
            pltpu.sync_copy(gb, agg_hbm.at[pl.ds(row_off + base, BN)])

        issue(0, idx0, g0, sem0)

        @pl.loop(0, (nb + 1) // 2)
        def _pair(p):
            m0 = 2 * p
            m1 = m0 + 1

            @pl.when(m1 < nb)
            def _():
                issue(m1, idx1, g1, sem1)

            finish(m0, idx0, g0, sem0)

            @pl.when(m0 + 2 < nb)
            def _():
                issue(m0 + 2, idx0, g0, sem0)

            @pl.when(m1 < nb)
            def _():
                finish(m1, idx1, g1, sem1)

    _phase(NBLK_NEWS, nidx_hbm, ent_hbm, 0)
    _phase(NBLK_ENT, neidx_hbm, all_hbm, N_NEWS)


NNZ_PER_TILE = NNZ // NW
NCHUNK = NNZ_PER_TILE // CB
URT = N_USERS // NS


@functools.partial(
    pl.kernel,
    out_type=jax.ShapeDtypeStruct((NC, N_USERS, Dp), jnp.float32),
    mesh=_mesh,
    scratch_types=[
        pltpu.VMEM((CB,), jnp.int32),
        pltpu.VMEM((CB,), jnp.int32),
        pltpu.VMEM((CB,), jnp.int32),
        pltpu.VMEM((CB,), jnp.int32),
        pltpu.VMEM((CB,), jnp.float32),
        pltpu.VMEM((CB,), jnp.float32),
        pltpu.VMEM((CB, Dp), jnp.float32),
        pltpu.VMEM((CB, Dp), jnp.float32),
        pltpu.VMEM_SHARED((N_USERS, Dp), jnp.float32),
        pltpu.SemaphoreType.DMA,
        pltpu.SemaphoreType.DMA,
        pltpu.SemaphoreType.DMA,
        pltpu.SemaphoreType.DMA,
    ],
)
def _user_kernel(node_hbm, cols_hbm, vals_hbm, rows_hbm, out_hbm,
                 idx0, idx1, row0, row1, val0, val1, g0, g1, acc_sh,
                 semg0, semg1, sems0, sems1):
    c = lax.axis_index("c")
    s = lax.axis_index("s")
    gid = s * NC + c
    base = gid * NNZ_PER_TILE

    @pl.loop(0, CB)
    def _z(r):
        for ci in range(NCH):
            g0[r, pl.ds(ci * L, L)] = jnp.zeros((L,), jnp.float32)

    pltpu.sync_copy(g0, acc_sh.at[pl.ds(s * URT, CB)])
    pltpu.sync_copy(g0, acc_sh.at[pl.ds(s * URT + CB, CB)])
    plsc.subcore_barrier()

    def issue(k, ib, rb, vb, gb, semg, sems, drain):
        if drain:
            pltpu.make_async_copy(node_hbm.at[pl.ds(0, CB)], gb, sems).wait()
        cbase = base + k * CB
        pltpu.sync_copy(cols_hbm.at[pl.ds(cbase, CB)], ib)
        pltpu.sync_copy(rows_hbm.at[pl.ds(cbase, CB)], rb)
        pltpu.sync_copy(vals_hbm.at[pl.ds(cbase, CB)], vb)
        pltpu.async_copy(node_hbm.at[ib], gb, semg)

    def finish(k, ib, rb, vb, gb, semg, sems):
        pltpu.make_async_copy(node_hbm.at[ib], gb, semg).wait()

        @pl.loop(0, CB // L)
        def _group(g):
            vvec = vb[pl.ds(g * L, L)]
            for j in range(L):
                r = g * L + j
                vv = vvec[j]
                for ci in range(NCH - 1):
                    sl = pl.ds(ci * L, L)
                    gb[r, sl] = gb[r, sl] * vv

        pltpu.async_copy(gb, acc_sh.at[rb], sems, add=True)

    issue(0, idx0, row0, val0, g0, semg0, sems0, drain=False)
    issue(1, idx1, row1, val1, g1, semg1, sems1, drain=False)

    @pl.loop(0, NCHUNK // 2)
    def _pair(p):
        k0 = 2 * p
        k1 = k0 + 1
        finish(k0, idx0, row0, val0, g0, semg0, sems0)

        @pl.when(k0 + 2 < NCHUNK)
        def _():
            issue(k0 + 2, idx0, row0, val0, g0, semg0, sems0, drain=True)

        finish(k1, idx1, row1, val1, g1, semg1, sems1)

        @pl.when(k1 + 2 < NCHUNK)
        def _():
            issue(k1 + 2, idx1, row1, val1, g1, semg1, sems1, drain=True)

    pltpu.make_async_copy(node_hbm.at[pl.ds(0, CB)], g0, sems0).wait()
    pltpu.make_async_copy(node_hbm.at[pl.ds(0, CB)], g1, sems1).wait()
    plsc.subcore_barrier()
    pltpu.sync_copy(acc_sh.at[pl.ds(s * URT, URT)],
                    out_hbm.at[c].at[pl.ds(s * URT, URT)])


@functools.partial(
    pl.kernel,
    out_type=jax.ShapeDtypeStruct((N_USERS, Dp), jnp.float32),
    mesh=_mesh,
    scratch_types=[
        pltpu.VMEM((UPW, Dp), jnp.float32),
        pltpu.VMEM((UPW, Dp), jnp.float32),
        pltpu.VMEM((UPW, Dp), jnp.float32),
    ],
)
def _combine_kernel(part_hbm, mod_hbm, out_hbm, a_v, b_v, m_v):
    w = _worker_id()
    ubase = w * UPW
    pltpu.sync_copy(part_hbm.at[0].at[pl.ds(ubase, UPW)], a_v)
    pltpu.sync_copy(part_hbm.at[1].at[pl.ds(ubase, UPW)], b_v)
    pltpu.sync_copy(mod_hbm.at[pl.ds(ubase, UPW)], m_v)

    @pl.loop(0, UPW)
    def _row(r):
        for ci in range(NCH - 1):
            sl = pl.ds(ci * L, L)
            m_v[r, sl] = (a_v[r, sl] + b_v[r, sl]) * m_v[r, sl]

    pltpu.sync_copy(m_v, out_hbm.at[pl.ds(ubase, UPW)])


def _mod_body(ue_ref, le_ref, da_ref, wt_ref, out_ref):
    score = jax.nn.softmax(
        jnp.dot(ue_ref[...], le_ref[...].T, preferred_element_type=jnp.float32),
        axis=1)
    dw = jnp.dot(jax.nn.softmax(da_ref[...], axis=-1), wt_ref[...],
                 preferred_element_type=jnp.float32)
    out_ref[...] = 1.0 + jnp.dot(score, dw, preferred_element_type=jnp.float32)


_PB = 1000


def _pad_body(a_ref, b_ref, ap_ref, bp_ref):
    z = jnp.zeros((_PB, Dp - D), jnp.float32)
    ap_ref[...] = jnp.concatenate([a_ref[...], z], axis=1)
    bp_ref[...] = jnp.concatenate([b_ref[...], z], axis=1)


def _finalize_body(agg_ref, bias_ref, nodep_ref, nodeout_ref):
    f = agg_ref[...] * (1.0 / N_NEIGH) + bias_ref[...]
    nodep_ref[...] = f
    nodeout_ref[...] = f[:, :D]


def _bias_body(all_ref, rel_ref, nr_ref, out_ref):
    i = pl.program_id(0)
    a = all_ref[...]
    rel = rel_ref[...]
    z = jnp.zeros((_PB, Dp - D), jnp.float32)

    @pl.when(i < N_NEWS // _PB)
    def _():
        out_ref[...] = jnp.concatenate([a + rel[0][None, :], z], axis=1)

    @pl.when(i >= N_NEWS // _PB)
    def _():
        nr = nr_ref[...]
        onehot = (nr[:, :, None] ==
                  lax.broadcasted_iota(jnp.int32, (_PB, N_NEIGH, N_REL), 2))
        cnt = jnp.sum(onehot.astype(jnp.float32), axis=1)
        relpart = jnp.dot(cnt, rel, preferred_element_type=jnp.float32)
        out_ref[...] = jnp.concatenate([a + relpart * (1.0 / N_NEIGH), z],
                                       axis=1)


def kernel(user_emb, all_embedding, entity_emb, relation_emb, latent_emb, weight,
           disen_weight_att, interact_vals, news_entities, news_relations,
           neigh_entities, neigh_relations, interact_rows, interact_cols):
    nidx = news_entities.reshape(NBLK_NEWS, BN, N_NEIGH).transpose(0, 2, 1).reshape(-1)
    neidx = neigh_entities.reshape(NBLK_ENT, BN, N_NEIGH).transpose(0, 2, 1).reshape(-1)

    wt_p = jnp.pad(weight, ((0, 0), (0, Dp - D)))

    mod = pl.pallas_call(
        _mod_body,
        out_shape=jax.ShapeDtypeStruct((N_USERS, Dp), jnp.float32),
    )(user_emb, latent_emb, disen_weight_att, wt_p)

    ent_p, all_p = pl.pallas_call(
        _pad_body,
        grid=(N_ENTITY // _PB,),
        in_specs=[pl.BlockSpec((_PB, D), lambda i: (i, 0)),
                  pl.BlockSpec((_PB, D), lambda i: (i, 0))],
        out_specs=[pl.BlockSpec((_PB, Dp), lambda i: (i, 0)),
                   pl.BlockSpec((_PB, Dp), lambda i: (i, 0))],
        out_shape=[jax.ShapeDtypeStruct((N_ENTITY, Dp), jnp.float32),
                   jax.ShapeDtypeStruct((N_ENTITY, Dp), jnp.float32)],
    )(entity_emb, all_embedding)

    nnews_blk = N_NEWS // _PB
    bias = pl.pallas_call(
        _bias_body,
        grid=(N_NODES // _PB,),
        in_specs=[
            pl.BlockSpec((_PB, D),
                         lambda i: (jnp.where(i < nnews_blk, i, i - nnews_blk), 0)),
            pl.BlockSpec((N_REL, D), lambda i: (0, 0)),
            pl.BlockSpec((_PB, N_NEIGH),
                         lambda i: (jnp.where(i < nnews_blk, 0, i - nnews_blk), 0)),
        ],
        out_specs=pl.BlockSpec((_PB, Dp), lambda i: (i, 0)),
        out_shape=jax.ShapeDtypeStruct((N_NODES, Dp), jnp.float32),
    )(all_embedding, relation_emb, neigh_relations)

    agg = _node_kernel(ent_p, all_p, nidx, neidx)

    node_p, node_out = pl.pallas_call(
        _finalize_body,
        grid=(N_NODES // _PB,),
        in_specs=[pl.BlockSpec((_PB, Dp), lambda i: (i, 0)),
                  pl.BlockSpec((_PB, Dp), lambda i: (i, 0))],
        out_specs=[pl.BlockSpec((_PB, Dp), lambda i: (i, 0)),
                   pl.BlockSpec((_PB, D), lambda i: (i, 0))],
        out_shape=[jax.ShapeDtypeStruct((N_NODES, Dp), jnp.float32),
                   jax.ShapeDtypeStruct((N_NODES, D), jnp.float32)],
    )(agg, bias)

    part = _user_kernel(node_p, interact_cols, interact_vals, interact_rows)
    user_p = _combine_kernel(part, mod)

    return (node_out, user_p[:, :D])
